# baseline (speedup 1.0000x reference)
import jax
import jax.numpy as jnp
from jax import lax
from jax.experimental import pallas as pl
from jax.experimental.pallas import tpu as pltpu
from jax.experimental.pallas import tpu_sc as plsc

_H = 128
_O = 128
_L = 16
_NW = _O // _L


def _sc_body(idx_hbm, table_hbm, w_hbm, b_hbm, out_hbm,
             idx_v, row_v, w_v, b_v, acc_v, sem):
    base = lax.axis_index("s") * _L
    cpw = pltpu.async_copy(w_hbm.at[pl.ds(base, _L)], w_v, sem)
    cpb = pltpu.async_copy(b_hbm.at[pl.ds(base, _L)], b_v, sem)
    pltpu.sync_copy(idx_hbm, idx_v)
    cpr = pltpu.async_copy(table_hbm.at[idx_v], row_v, sem)
    cpw.wait()
    cpb.wait()
    cpr.wait()
    rs = [row_v[0, pl.ds(kb * _L, _L)] for kb in range(_H // _L)]
    lane = lax.iota(jnp.int32, _L)

    def jbody(jl, out):
        acc = rs[0] * w_v[jl, pl.ds(0, _L)]
        for kb in range(1, _H // _L):
            acc = acc + rs[kb] * w_v[jl, pl.ds(kb * _L, _L)]
        parts = [acc[l] for l in range(_L)]
        while len(parts) > 1:
            parts = [parts[i] + parts[i + 1]
                     for i in range(0, len(parts), 2)]
        return jnp.where(lane == jl, out + parts[0], out)

    acc_v[...] = lax.fori_loop(0, _L, jbody, b_v[...])
    pltpu.sync_copy(acc_v, out_hbm.at[pl.ds(base, _L)])


def kernel(indices, table, W, b):
    out = pl.kernel(
        _sc_body,
        out_type=jax.ShapeDtypeStruct((_O,), jnp.float32),
        mesh=plsc.VectorSubcoreMesh(core_axis_name="c", subcore_axis_name="s",
                                    num_cores=1, num_subcores=_NW),
        scratch_types=[
            pltpu.VMEM((1,), jnp.int32),
            pltpu.VMEM((1, _H), jnp.float32),
            pltpu.VMEM((_L, _H), jnp.float32),
            pltpu.VMEM((_L,), jnp.float32),
            pltpu.VMEM((_L,), jnp.float32),
            pltpu.SemaphoreType.DMA,
        ],
    )(indices.astype(jnp.int32), table, W, b)
    return out.reshape(1, _O)
